# baseline (speedup 1.0000x reference)
import jax
import jax.numpy as jnp
from jax import lax
from jax.experimental import pallas as pl

TOPK = 8


def _proj_body(q_ref, w_ref, b_ref, out_ref):
    q = q_ref[...]
    w = w_ref[...]
    acc = lax.dot_general(q, w, (((1,), (1,)), ((), ())),
                          preferred_element_type=jnp.float32)
    out_ref[...] = acc + b_ref[...]


def _dist_topk_body(qp_ref, ctx_ref, qsq_ref, csq_ref, val_ref, idx_ref):
    nb = qp_ref.shape[0]
    for b in range(nb):
        qb = qp_ref[b]
        cb = ctx_ref[b]
        D = qb.shape[1]
        nk = 4
        dk = D // nk
        cross = None
        for i in range(nk):
            s = slice(i * dk, (i + 1) * dk)
            p = lax.dot_general(qb[:, s], cb[:, s], (((1,), (1,)), ((), ())),
                                preferred_element_type=jnp.float32)
            cross = p if cross is None else cross + p
        q_sq = qsq_ref[b]
        c_sq = csq_ref[b]
        d2 = q_sq + c_sq - 2.0 * cross
        dist = jnp.sqrt(jnp.maximum(d2, 1e-12))

        Q, C = dist.shape
        col = lax.broadcasted_iota(jnp.int32, (Q, C), 1)
        vals = []
        idxs = []
        for _ in range(TOPK):
            m = jnp.min(dist, axis=1, keepdims=True)
            hit = dist == m
            amin = jnp.min(jnp.where(hit, col, C), axis=1, keepdims=True)
            vals.append(m)
            idxs.append(amin)
            dist = jnp.where(col == amin, jnp.inf, dist)
        val_ref[b] = jnp.concatenate(vals, axis=1)
        idx_ref[b] = jnp.concatenate(idxs, axis=1)


@jax.jit
def kernel(query_embeddings, context_embeddings, W_q, b_q):
    B, Q, D = query_embeddings.shape
    C = context_embeddings.shape[1]

    q2 = query_embeddings.reshape(B * Q, D)
    qp = pl.pallas_call(
        _proj_body,
        out_shape=jax.ShapeDtypeStruct((B * Q, D), jnp.float32),
    )(q2, W_q, b_q.reshape(1, D))
    qp3 = qp.reshape(B, Q, D)

    qp_x = jnp.einsum('bqd,ed->bqe', query_embeddings, W_q) + b_q
    q_sq = jnp.sum(qp_x * qp_x, axis=-1, keepdims=True)
    c_sq = jnp.sum(context_embeddings * context_embeddings, axis=-1)
    c_sq3 = c_sq[:, None, :]

    NB = 2
    grid = (B // NB,)
    val, idx = pl.pallas_call(
        _dist_topk_body,
        grid=grid,
        in_specs=[
            pl.BlockSpec((NB, Q, D), lambda b: (b, 0, 0)),
            pl.BlockSpec((NB, C, D), lambda b: (b, 0, 0)),
            pl.BlockSpec((NB, Q, 1), lambda b: (b, 0, 0)),
            pl.BlockSpec((NB, 1, C), lambda b: (b, 0, 0)),
        ],
        out_specs=[
            pl.BlockSpec((NB, Q, TOPK), lambda b: (b, 0, 0)),
            pl.BlockSpec((NB, Q, TOPK), lambda b: (b, 0, 0)),
        ],
        out_shape=[
            jax.ShapeDtypeStruct((B, Q, TOPK), jnp.float32),
            jax.ShapeDtypeStruct((B, Q, TOPK), jnp.int32),
        ],
    )(qp3, context_embeddings, q_sq, c_sq3)
    return val, idx
